# Initial kernel scaffold; baseline (speedup 1.0000x reference)
import jax
import jax.numpy as jnp
from jax.experimental import pallas as pl


def kernel(values, table, W, b):
    raise NotImplementedError("write your pallas kernel here")


# speedup vs baseline: 51.5108x; 51.5108x over previous
import functools

import jax
import jax.numpy as jnp
from jax import lax
from jax.experimental import pallas as pl
from jax.experimental.pallas import tpu as pltpu
from jax.experimental.pallas import tpu_sc as plsc

_NUM_ROWS = 100000
_DIM = 64
_TOTAL_IDX = 26 * 4096 * 20

_NC = 2
_NS = 16
_L = 16
_NW = _NC * _NS

_PER_W = _TOTAL_IDX // _NW
_NCHUNK = 8
_CHUNK = _PER_W // _NCHUNK
_GROUPS = _CHUNK // _L


_TC_BLK = 10000


def _row_scores_body(tbl_ref, w_ref, o_ref):
    o_ref[...] = jnp.sum(tbl_ref[...] * w_ref[...], axis=1, keepdims=True)


def _row_scores(table, W):
    return pl.pallas_call(
        _row_scores_body,
        grid=(_NUM_ROWS // _TC_BLK,),
        in_specs=[
            pl.BlockSpec((_TC_BLK, _DIM), lambda i: (i, 0)),
            pl.BlockSpec((1, _DIM), lambda i: (0, 0)),
        ],
        out_specs=pl.BlockSpec((_TC_BLK, 1), lambda i: (i, 0)),
        out_shape=jax.ShapeDtypeStruct((_NUM_ROWS, 1), jnp.float32),
    )(table, W)


_sc_mesh = plsc.VectorSubcoreMesh(core_axis_name="c", subcore_axis_name="s")


@functools.partial(
    pl.kernel,
    out_type=jax.ShapeDtypeStruct((_NW, _L), jnp.float32),
    mesh=_sc_mesh,
    scratch_types=[
        pltpu.VMEM((_NUM_ROWS,), jnp.float32),
        pltpu.VMEM((_CHUNK,), jnp.int32),
        pltpu.VMEM((_CHUNK,), jnp.int32),
        pltpu.VMEM((_L,), jnp.float32),
        pltpu.SemaphoreType.DMA,
        pltpu.SemaphoreType.DMA,
    ],
    compiler_params=pltpu.CompilerParams(needs_layout_passes=False),
)
def _sc_gather_sum(t_hbm, idx_hbm, out_hbm,
                   t_v, idx_a, idx_b, stage_v,
                   sem_a, sem_b):
    wid = lax.axis_index("s") * _NC + lax.axis_index("c")
    base = wid * _PER_W

    bufs = (idx_a, idx_b)
    sems = (sem_a, sem_b)

    t_cp = pltpu.async_copy(t_hbm, t_v, sem_b)
    cp0 = pltpu.async_copy(idx_hbm.at[pl.ds(base, _CHUNK)], idx_a, sem_a)
    t_cp.wait()
    cp0.wait()

    def chunk_sum(buf, acc):
        def body(g, acc):
            idx16 = buf[pl.ds(g * _L, _L)]
            return acc + plsc.load_gather(t_v, [idx16])
        return lax.fori_loop(0, _GROUPS, body, acc, unroll=4)

    acc = jnp.zeros((_L,), jnp.float32)
    for c in range(_NCHUNK):
        if c + 1 < _NCHUNK:
            nxt = pltpu.async_copy(
                idx_hbm.at[pl.ds(base + (c + 1) * _CHUNK, _CHUNK)],
                bufs[(c + 1) % 2], sems[(c + 1) % 2])
        acc = chunk_sum(bufs[c % 2], acc)
        if c + 1 < _NCHUNK:
            nxt.wait()

    stage_v[...] = acc
    pltpu.sync_copy(stage_v, out_hbm.at[wid])


def kernel(values, table, W, b):
    idx = values.astype(jnp.int32).reshape(-1)
    t = _row_scores(table, W).reshape(-1)
    partials = _sc_gather_sum(t, idx)
    return jnp.sum(partials).reshape(1) + b
